# Initial kernel scaffold; baseline (speedup 1.0000x reference)
import math

import jax
import jax.numpy as jnp
from jax.experimental import pallas as pl
from jax.experimental.pallas import tpu as pltpu

_B, _S, _D = 16, 2048, 1024
_TEMP = 0.07
_P = 64
_BETA = 0.4
_TS = 256


def _prep_body(xh, xi, wiq, biq, whq, bhq, wk, wfc2, qt_o, u_o):
    img_q = jnp.dot(xi[...], wiq[...], preferred_element_type=jnp.float32) + biq[...]
    head_q = jnp.dot(xh[...], whq[...], preferred_element_type=jnp.float32) + bhq[...]
    qc = img_q * _BETA + head_q * (1.0 - _BETA)
    scale = 1.0 / (math.sqrt(_D) * _TEMP)
    qt = jax.lax.dot_general(qc, wk[...], (((1,), (1,)), ((), ())),
                             preferred_element_type=jnp.float32) * scale
    qt_o[...] = qt
    u_o[...] = jax.lax.dot_general(qt, wfc2[...], (((1,), (1,)), ((), ())),
                                   preferred_element_type=jnp.float32)


def _main_body(xt_ref, xr_ref, qt_ref, u_ref, mask_ref, w1t_ref, w1r_ref, b1_ref,
               scores_ref, h_ref):
    xt = xt_ref[0]
    xr = xr_ref[0]
    z = (jnp.dot(xt, w1t_ref[...], preferred_element_type=jnp.float32)
         + jnp.dot(xr, w1r_ref[...], preferred_element_type=jnp.float32)
         + b1_ref[...])
    h = jnp.where(z >= 0.0, z, 0.01 * z)
    h_ref[0] = h
    s = (jnp.sum(xt * qt_ref[...], axis=1)
         + jnp.sum(h * u_ref[...], axis=1))
    s = jnp.where(mask_ref[0] < 0.01, -jnp.inf, s)
    scores_ref[0] = s


def _topk_body(scores_ref, w_ref, sw_ref):
    s = scores_ref[...]
    m = jnp.max(s, axis=1, keepdims=True)
    e = jnp.exp(s - m)
    denom = jnp.sum(e, axis=1, keepdims=True)
    cur = s
    col = jax.lax.broadcasted_iota(jnp.int32, (_B, _S), 1)
    keep = jnp.zeros((_B, _S), jnp.float32)
    for _ in range(_P):
        mx = jnp.max(cur, axis=1, keepdims=True)
        hit = cur == mx
        first = jnp.min(jnp.where(hit, col, _S), axis=1, keepdims=True)
        sel = col == first
        keep = jnp.where(sel, 1.0, keep)
        cur = jnp.where(sel, -jnp.inf, cur)
    w = keep * e / denom
    w_ref[...] = w
    sw = jnp.sum(w, axis=1, keepdims=True)
    sw_ref[...] = jnp.broadcast_to(sw, (_B, 128))


def _pass2_body(w_ref, xt_ref, h_ref, t1_ref, t2_ref):
    st = pl.program_id(1)
    w = w_ref[...]
    t1 = jnp.dot(w, xt_ref[0], preferred_element_type=jnp.float32)
    t2 = jnp.dot(w, h_ref[0], preferred_element_type=jnp.float32)

    @pl.when(st == 0)
    def _init():
        t1_ref[...] = t1
        t2_ref[...] = t2

    @pl.when(st != 0)
    def _acc():
        t1_ref[...] += t1
        t2_ref[...] += t2


def _finale_body(t1, t2, sw, xh, xi, wfc2, bfc2, wv, bv, wproj, bproj,
                 wft, bft, wfi, bfi, wftr, bftr, wgt, bgt, wgi, bgi, out_ref):
    sw_col = sw[:, :1]
    f32 = jnp.float32
    reltail = (t1[...] + jnp.dot(t2[...], wfc2[...], preferred_element_type=f32)
               + sw_col * bfc2[...])
    ctx = jnp.dot(reltail, wv[...], preferred_element_type=f32) + sw_col * bv[...]
    x_triple = jnp.dot(ctx, wproj[...], preferred_element_type=f32) + bproj[...]
    triple_out = jnp.dot(x_triple, wftr[...], preferred_element_type=f32) + bftr[...]
    text_out = jnp.dot(xh[...], wft[...], preferred_element_type=f32) + bft[...]
    img_out = jnp.dot(xi[...], wfi[...], preferred_element_type=f32) + bfi[...]
    tg = jax.nn.sigmoid(jnp.dot(xh[...], wgt[...], preferred_element_type=f32) + bgt[...])
    ig = jax.nn.sigmoid(jnp.dot(xi[...], wgi[...], preferred_element_type=f32) + bgi[...])
    out_ref[...] = triple_out + text_out * tg + img_out * ig


def kernel(x_head, x_rel, x_tail, x_mask, x_img, W_fc1, b_fc1, W_fc2, b_fc2,
           W_k, b_k, W_v, b_v, W_proj, b_proj, W_iq, b_iq, W_hq, b_hq,
           W_ft, b_ft, W_fi, b_fi, W_ftr, b_ftr, W_gt, b_gt, W_gi, b_gi):
    f32 = jnp.float32
    r2 = lambda b: b.reshape(1, _D)
    w1t = W_fc1[:_D]
    w1r = W_fc1[_D:]

    qt, u = pl.pallas_call(
        _prep_body,
        out_shape=[jax.ShapeDtypeStruct((_B, _D), f32)] * 2,
    )(x_head, x_img, W_iq, r2(b_iq), W_hq, r2(b_hq), W_k, W_fc2)

    nst = _S // _TS
    scores, h = pl.pallas_call(
        _main_body,
        grid=(_B, nst),
        in_specs=[
            pl.BlockSpec((1, _TS, _D), lambda b, s: (b, s, 0)),
            pl.BlockSpec((1, _TS, _D), lambda b, s: (b, s, 0)),
            pl.BlockSpec((1, _D), lambda b, s: (b, 0)),
            pl.BlockSpec((1, _D), lambda b, s: (b, 0)),
            pl.BlockSpec((1, _TS), lambda b, s: (b, s)),
            pl.BlockSpec((_D, _D), lambda b, s: (0, 0)),
            pl.BlockSpec((_D, _D), lambda b, s: (0, 0)),
            pl.BlockSpec((1, _D), lambda b, s: (0, 0)),
        ],
        out_specs=[
            pl.BlockSpec((1, _TS), lambda b, s: (b, s)),
            pl.BlockSpec((1, _TS, _D), lambda b, s: (b, s, 0)),
        ],
        out_shape=[
            jax.ShapeDtypeStruct((_B, _S), f32),
            jax.ShapeDtypeStruct((_B, _S, _D), f32),
        ],
    )(x_tail, x_rel, qt, u, x_mask, w1t, w1r, r2(b_fc1))

    w, sw = pl.pallas_call(
        _topk_body,
        out_shape=[
            jax.ShapeDtypeStruct((_B, _S), f32),
            jax.ShapeDtypeStruct((_B, 128), f32),
        ],
    )(scores)

    t1, t2 = pl.pallas_call(
        _pass2_body,
        grid=(_B, nst),
        in_specs=[
            pl.BlockSpec((1, _TS), lambda b, s: (b, s)),
            pl.BlockSpec((1, _TS, _D), lambda b, s: (b, s, 0)),
            pl.BlockSpec((1, _TS, _D), lambda b, s: (b, s, 0)),
        ],
        out_specs=[
            pl.BlockSpec((1, _D), lambda b, s: (b, 0)),
            pl.BlockSpec((1, _D), lambda b, s: (b, 0)),
        ],
        out_shape=[jax.ShapeDtypeStruct((_B, _D), f32)] * 2,
    )(w, x_tail, h)

    out = pl.pallas_call(
        _finale_body,
        out_shape=jax.ShapeDtypeStruct((_B, _D), f32),
    )(t1, t2, sw, x_head, x_img, W_fc2, r2(b_fc2), W_v, r2(b_v),
      W_proj, r2(b_proj), W_ft, r2(b_ft), W_fi, r2(b_fi),
      W_ftr, r2(b_ftr), W_gt, r2(b_gt), W_gi, r2(b_gi))
    return out


# speedup vs baseline: 1.5361x; 1.5361x over previous
import math

import jax
import jax.numpy as jnp
from jax.experimental import pallas as pl
from jax.experimental.pallas import tpu as pltpu

_B, _S, _D = 16, 2048, 1024
_TEMP = 0.07
_P = 64
_BETA = 0.4
_TS = 256


def _prep_body(xh, xi, wiq, biq, whq, bhq, wk, wfc2, qt_o, u_o):
    img_q = jnp.dot(xi[...], wiq[...], preferred_element_type=jnp.float32) + biq[...]
    head_q = jnp.dot(xh[...], whq[...], preferred_element_type=jnp.float32) + bhq[...]
    qc = img_q * _BETA + head_q * (1.0 - _BETA)
    scale = 1.0 / (math.sqrt(_D) * _TEMP)
    qt = jax.lax.dot_general(qc, wk[...], (((1,), (1,)), ((), ())),
                             preferred_element_type=jnp.float32) * scale
    qt_o[...] = qt
    u_o[...] = jax.lax.dot_general(qt, wfc2[...], (((1,), (1,)), ((), ())),
                                   preferred_element_type=jnp.float32)


def _main_body(xt_ref, xr_ref, qt_ref, u_ref, mask_ref, w1t_ref, w1r_ref, b1_ref,
               scores_ref, h_ref):
    xt = xt_ref[0]
    xr = xr_ref[0]
    z = (jnp.dot(xt, w1t_ref[...], preferred_element_type=jnp.float32)
         + jnp.dot(xr, w1r_ref[...], preferred_element_type=jnp.float32)
         + b1_ref[...])
    h = jnp.where(z >= 0.0, z, 0.01 * z)
    h_ref[0] = h
    s = (jnp.sum(xt * qt_ref[0], axis=1)
         + jnp.sum(h * u_ref[0], axis=1))
    s = jnp.where(mask_ref[0, 0] < 0.01, -jnp.inf, s)
    scores_ref[0, 0] = s


def _topk_body(scores_ref, w_ref, sw_ref):
    s = scores_ref[:, 0, :]
    m = jnp.max(s, axis=1, keepdims=True)
    e = jnp.exp(s - m)
    denom = jnp.sum(e, axis=1, keepdims=True)
    cur = s
    col = jax.lax.broadcasted_iota(jnp.int32, (_B, _S), 1)
    keep = jnp.zeros((_B, _S), jnp.float32)
    for _ in range(_P):
        mx = jnp.max(cur, axis=1, keepdims=True)
        hit = cur == mx
        first = jnp.min(jnp.where(hit, col, _S), axis=1, keepdims=True)
        sel = col == first
        keep = jnp.where(sel, 1.0, keep)
        cur = jnp.where(sel, -jnp.inf, cur)
    w = keep * e / denom
    w_ref[:, 0, :] = w
    sw = jnp.sum(w, axis=1, keepdims=True)
    sw_ref[...] = jnp.broadcast_to(sw, (_B, 128))


def _pass2_body(w_ref, xt_ref, h_ref, t1_ref, t2_ref):
    st = pl.program_id(1)
    w = w_ref[0]
    t1 = jnp.dot(w, xt_ref[0], preferred_element_type=jnp.float32)
    t2 = jnp.dot(w, h_ref[0], preferred_element_type=jnp.float32)

    @pl.when(st == 0)
    def _init():
        t1_ref[0] = t1
        t2_ref[0] = t2

    @pl.when(st != 0)
    def _acc():
        t1_ref[0] += t1
        t2_ref[0] += t2


def _finale_body(t1_ref, t2_ref, sw, xh, xi, wfc2, bfc2, wv, bv, wproj, bproj,
                 wft, bft, wfi, bfi, wftr, bftr, wgt, bgt, wgi, bgi, out_ref):
    sw_col = sw[:, :1]
    f32 = jnp.float32
    t1 = t1_ref[:, 0, :]
    t2 = t2_ref[:, 0, :]
    reltail = (t1 + jnp.dot(t2, wfc2[...], preferred_element_type=f32)
               + sw_col * bfc2[...])
    ctx = jnp.dot(reltail, wv[...], preferred_element_type=f32) + sw_col * bv[...]
    x_triple = jnp.dot(ctx, wproj[...], preferred_element_type=f32) + bproj[...]
    triple_out = jnp.dot(x_triple, wftr[...], preferred_element_type=f32) + bftr[...]
    text_out = jnp.dot(xh[...], wft[...], preferred_element_type=f32) + bft[...]
    img_out = jnp.dot(xi[...], wfi[...], preferred_element_type=f32) + bfi[...]
    tg = jax.nn.sigmoid(jnp.dot(xh[...], wgt[...], preferred_element_type=f32) + bgt[...])
    ig = jax.nn.sigmoid(jnp.dot(xi[...], wgi[...], preferred_element_type=f32) + bgi[...])
    out_ref[...] = triple_out + text_out * tg + img_out * ig


def kernel(x_head, x_rel, x_tail, x_mask, x_img, W_fc1, b_fc1, W_fc2, b_fc2,
           W_k, b_k, W_v, b_v, W_proj, b_proj, W_iq, b_iq, W_hq, b_hq,
           W_ft, b_ft, W_fi, b_fi, W_ftr, b_ftr, W_gt, b_gt, W_gi, b_gi):
    f32 = jnp.float32
    r2 = lambda b: b.reshape(1, _D)
    w1t = W_fc1[:_D]
    w1r = W_fc1[_D:]

    qt, u = pl.pallas_call(
        _prep_body,
        out_shape=[jax.ShapeDtypeStruct((_B, _D), f32)] * 2,
    )(x_head, x_img, W_iq, r2(b_iq), W_hq, r2(b_hq), W_k, W_fc2)

    nst = _S // _TS
    qt3 = qt.reshape(_B, 1, _D)
    u3 = u.reshape(_B, 1, _D)
    mask3 = x_mask.reshape(_B, 1, _S)
    scores, h = pl.pallas_call(
        _main_body,
        grid=(_B, nst),
        in_specs=[
            pl.BlockSpec((1, _TS, _D), lambda b, s: (b, s, 0)),
            pl.BlockSpec((1, _TS, _D), lambda b, s: (b, s, 0)),
            pl.BlockSpec((1, 1, _D), lambda b, s: (b, 0, 0)),
            pl.BlockSpec((1, 1, _D), lambda b, s: (b, 0, 0)),
            pl.BlockSpec((1, 1, _TS), lambda b, s: (b, 0, s)),
            pl.BlockSpec((_D, _D), lambda b, s: (0, 0)),
            pl.BlockSpec((_D, _D), lambda b, s: (0, 0)),
            pl.BlockSpec((1, _D), lambda b, s: (0, 0)),
        ],
        out_specs=[
            pl.BlockSpec((1, 1, _TS), lambda b, s: (b, 0, s)),
            pl.BlockSpec((1, _TS, _D), lambda b, s: (b, s, 0)),
        ],
        out_shape=[
            jax.ShapeDtypeStruct((_B, 1, _S), f32),
            jax.ShapeDtypeStruct((_B, _S, _D), f32),
        ],
    )(x_tail, x_rel, qt3, u3, mask3, w1t, w1r, r2(b_fc1))

    w, sw = pl.pallas_call(
        _topk_body,
        out_shape=[
            jax.ShapeDtypeStruct((_B, 1, _S), f32),
            jax.ShapeDtypeStruct((_B, 128), f32),
        ],
    )(scores)

    t1, t2 = pl.pallas_call(
        _pass2_body,
        grid=(_B, nst),
        in_specs=[
            pl.BlockSpec((1, 1, _TS), lambda b, s: (b, 0, s)),
            pl.BlockSpec((1, _TS, _D), lambda b, s: (b, s, 0)),
            pl.BlockSpec((1, _TS, _D), lambda b, s: (b, s, 0)),
        ],
        out_specs=[
            pl.BlockSpec((1, 1, _D), lambda b, s: (b, 0, 0)),
            pl.BlockSpec((1, 1, _D), lambda b, s: (b, 0, 0)),
        ],
        out_shape=[jax.ShapeDtypeStruct((_B, 1, _D), f32)] * 2,
    )(w, x_tail, h)

    out = pl.pallas_call(
        _finale_body,
        out_shape=jax.ShapeDtypeStruct((_B, _D), f32),
    )(t1, t2, sw, x_head, x_img, W_fc2, r2(b_fc2), W_v, r2(b_v),
      W_proj, r2(b_proj), W_ft, r2(b_ft), W_fi, r2(b_fi),
      W_ftr, r2(b_ftr), W_gt, r2(b_gt), W_gi, r2(b_gi))
    return out


# speedup vs baseline: 2.0181x; 1.3138x over previous
import functools
import math

import jax
import jax.numpy as jnp
from jax import lax
from jax.experimental import pallas as pl
from jax.experimental.pallas import tpu as pltpu
from jax.experimental.pallas import tpu_sc as plsc

_B, _S, _D = 16, 2048, 1024
_TEMP = 0.07
_P = 64
_BETA = 0.4
_TS = 256

_SC_INFO = plsc.get_sparse_core_info()
_NW = _SC_INFO.num_cores * _SC_INFO.num_subcores
_BP = _B * _P
_BPW = _BP // _NW


def _prep_body(xh, xi, wiq, biq, whq, bhq, wk, wfc2, qt_o, u_o):
    img_q = jnp.dot(xi[...], wiq[...], preferred_element_type=jnp.float32) + biq[...]
    head_q = jnp.dot(xh[...], whq[...], preferred_element_type=jnp.float32) + bhq[...]
    qc = img_q * _BETA + head_q * (1.0 - _BETA)
    scale = 1.0 / (math.sqrt(_D) * _TEMP)
    qt = jax.lax.dot_general(qc, wk[...], (((1,), (1,)), ((), ())),
                             preferred_element_type=jnp.float32) * scale
    qt_o[...] = qt
    u_o[...] = jax.lax.dot_general(qt, wfc2[...], (((1,), (1,)), ((), ())),
                                   preferred_element_type=jnp.float32)


def _main_body(xt_ref, xr_ref, qt_ref, u_ref, mask_ref, w1t_ref, w1r_ref, b1_ref,
               scores_ref):
    xt = xt_ref[0]
    xr = xr_ref[0]
    z = (jnp.dot(xt, w1t_ref[...], preferred_element_type=jnp.float32)
         + jnp.dot(xr, w1r_ref[...], preferred_element_type=jnp.float32)
         + b1_ref[...])
    h = jnp.where(z >= 0.0, z, 0.01 * z)
    s = (jnp.sum(xt * qt_ref[0], axis=1)
         + jnp.sum(h * u_ref[0], axis=1))
    s = jnp.where(mask_ref[0, 0] < 0.01, -jnp.inf, s)
    scores_ref[0, 0] = s


def _topk_body(scores_ref, idx_ref, w_ref):
    s = scores_ref[:, 0, :]
    m = jnp.max(s, axis=1, keepdims=True)
    e = jnp.exp(s - m)
    denom = jnp.sum(e, axis=1, keepdims=True)
    cur = s
    col = jax.lax.broadcasted_iota(jnp.int32, (_B, _S), 1)
    lane = jax.lax.broadcasted_iota(jnp.int32, (_B, _P), 1)
    row_off = jax.lax.broadcasted_iota(jnp.int32, (_B, _P), 0) * _S
    idx_acc = jnp.zeros((_B, _P), jnp.int32)
    val_acc = jnp.zeros((_B, _P), jnp.float32)
    for i in range(_P):
        mx = jnp.max(cur, axis=1, keepdims=True)
        hit = cur == mx
        first = jnp.min(jnp.where(hit, col, _S), axis=1, keepdims=True)
        sel = col == first
        val = jnp.sum(jnp.where(sel, e, 0.0), axis=1, keepdims=True)
        idx_acc = jnp.where(lane == i, first, idx_acc)
        val_acc = jnp.where(lane == i, val, val_acc)
        cur = jnp.where(sel, -jnp.inf, cur)
    idx_ref[...] = idx_acc + row_off
    w_ref[:, 0, :] = val_acc / denom


def _sc_gather_body(xt_hbm, xr_hbm, idx_hbm, outt_hbm, outr_hbm,
                    idx_v, rows_v, sem):
    nc = _SC_INFO.num_cores
    wid = lax.axis_index("s") * nc + lax.axis_index("c")
    base = wid * _BPW
    pltpu.sync_copy(idx_hbm.at[pl.ds(base, _BPW)], idx_v)
    pltpu.async_copy(xt_hbm.at[idx_v], rows_v, sem).wait()
    pltpu.sync_copy(rows_v, outt_hbm.at[pl.ds(base, _BPW)])
    pltpu.async_copy(xr_hbm.at[idx_v], rows_v, sem).wait()
    pltpu.sync_copy(rows_v, outr_hbm.at[pl.ds(base, _BPW)])


def _tk_body(xt_ref, xr_ref, w_ref, w1t_ref, w1r_ref, b1_ref, t1_ref, t2_ref):
    xt = xt_ref[0]
    xr = xr_ref[0]
    z = (jnp.dot(xt, w1t_ref[...], preferred_element_type=jnp.float32)
         + jnp.dot(xr, w1r_ref[...], preferred_element_type=jnp.float32)
         + b1_ref[...])
    h = jnp.where(z >= 0.0, z, 0.01 * z)
    w = w_ref[0]
    t1_ref[0] = jnp.dot(w, xt, preferred_element_type=jnp.float32)
    t2_ref[0] = jnp.dot(w, h, preferred_element_type=jnp.float32)


def _finale_body(t1_ref, t2_ref, w_ref, xh, xi, wfc2, bfc2, wv, bv, wproj, bproj,
                 wft, bft, wfi, bfi, wftr, bftr, wgt, bgt, wgi, bgi, out_ref):
    f32 = jnp.float32
    sw_col = jnp.sum(w_ref[:, 0, :], axis=1, keepdims=True)
    t1 = t1_ref[:, 0, :]
    t2 = t2_ref[:, 0, :]
    reltail = (t1 + jnp.dot(t2, wfc2[...], preferred_element_type=f32)
               + sw_col * bfc2[...])
    ctx = jnp.dot(reltail, wv[...], preferred_element_type=f32) + sw_col * bv[...]
    x_triple = jnp.dot(ctx, wproj[...], preferred_element_type=f32) + bproj[...]
    triple_out = jnp.dot(x_triple, wftr[...], preferred_element_type=f32) + bftr[...]
    text_out = jnp.dot(xh[...], wft[...], preferred_element_type=f32) + bft[...]
    img_out = jnp.dot(xi[...], wfi[...], preferred_element_type=f32) + bfi[...]
    tg = jax.nn.sigmoid(jnp.dot(xh[...], wgt[...], preferred_element_type=f32) + bgt[...])
    ig = jax.nn.sigmoid(jnp.dot(xi[...], wgi[...], preferred_element_type=f32) + bgi[...])
    out_ref[...] = triple_out + text_out * tg + img_out * ig


def kernel(x_head, x_rel, x_tail, x_mask, x_img, W_fc1, b_fc1, W_fc2, b_fc2,
           W_k, b_k, W_v, b_v, W_proj, b_proj, W_iq, b_iq, W_hq, b_hq,
           W_ft, b_ft, W_fi, b_fi, W_ftr, b_ftr, W_gt, b_gt, W_gi, b_gi):
    f32 = jnp.float32
    r2 = lambda b: b.reshape(1, _D)
    w1t = W_fc1[:_D]
    w1r = W_fc1[_D:]

    qt, u = pl.pallas_call(
        _prep_body,
        out_shape=[jax.ShapeDtypeStruct((_B, _D), f32)] * 2,
    )(x_head, x_img, W_iq, r2(b_iq), W_hq, r2(b_hq), W_k, W_fc2)

    nst = _S // _TS
    qt3 = qt.reshape(_B, 1, _D)
    u3 = u.reshape(_B, 1, _D)
    mask3 = x_mask.reshape(_B, 1, _S)
    scores = pl.pallas_call(
        _main_body,
        grid=(_B, nst),
        in_specs=[
            pl.BlockSpec((1, _TS, _D), lambda b, s: (b, s, 0)),
            pl.BlockSpec((1, _TS, _D), lambda b, s: (b, s, 0)),
            pl.BlockSpec((1, 1, _D), lambda b, s: (b, 0, 0)),
            pl.BlockSpec((1, 1, _D), lambda b, s: (b, 0, 0)),
            pl.BlockSpec((1, 1, _TS), lambda b, s: (b, 0, s)),
            pl.BlockSpec((_D, _D), lambda b, s: (0, 0)),
            pl.BlockSpec((_D, _D), lambda b, s: (0, 0)),
            pl.BlockSpec((1, _D), lambda b, s: (0, 0)),
        ],
        out_specs=pl.BlockSpec((1, 1, _TS), lambda b, s: (b, 0, s)),
        out_shape=jax.ShapeDtypeStruct((_B, 1, _S), f32),
    )(x_tail, x_rel, qt3, u3, mask3, w1t, w1r, r2(b_fc1))

    idx, w = pl.pallas_call(
        _topk_body,
        out_shape=[
            jax.ShapeDtypeStruct((_B, _P), jnp.int32),
            jax.ShapeDtypeStruct((_B, 1, _P), f32),
        ],
    )(scores)

    xt_flat = x_tail.reshape(_B * _S, _D)
    xr_flat = x_rel.reshape(_B * _S, _D)
    gidx = idx.reshape(_BP)
    sc_gather = pl.kernel(
        _sc_gather_body,
        mesh=plsc.VectorSubcoreMesh(core_axis_name="c", subcore_axis_name="s"),
        out_type=[jax.ShapeDtypeStruct((_BP, _D), f32)] * 2,
        scratch_types=[
            pltpu.VMEM((_BPW,), jnp.int32),
            pltpu.VMEM((_BPW, _D), f32),
            pltpu.SemaphoreType.DMA,
        ],
    )
    xt_top, xr_top = sc_gather(xt_flat, xr_flat, gidx)

    t1, t2 = pl.pallas_call(
        _tk_body,
        grid=(_B,),
        in_specs=[
            pl.BlockSpec((1, _P, _D), lambda b: (b, 0, 0)),
            pl.BlockSpec((1, _P, _D), lambda b: (b, 0, 0)),
            pl.BlockSpec((1, 1, _P), lambda b: (b, 0, 0)),
            pl.BlockSpec((_D, _D), lambda b: (0, 0)),
            pl.BlockSpec((_D, _D), lambda b: (0, 0)),
            pl.BlockSpec((1, _D), lambda b: (0, 0)),
        ],
        out_specs=[
            pl.BlockSpec((1, 1, _D), lambda b: (b, 0, 0)),
            pl.BlockSpec((1, 1, _D), lambda b: (b, 0, 0)),
        ],
        out_shape=[jax.ShapeDtypeStruct((_B, 1, _D), f32)] * 2,
    )(xt_top.reshape(_B, _P, _D), xr_top.reshape(_B, _P, _D), w, w1t, w1r, r2(b_fc1))

    out = pl.pallas_call(
        _finale_body,
        out_shape=jax.ShapeDtypeStruct((_B, _D), f32),
    )(t1, t2, w, x_head, x_img, W_fc2, r2(b_fc2), W_v, r2(b_v),
      W_proj, r2(b_proj), W_ft, r2(b_ft), W_fi, r2(b_fi),
      W_ftr, r2(b_ftr), W_gt, r2(b_gt), W_gi, r2(b_gi))
    return out
